# baseline (speedup 1.0000x reference)
import functools

import jax
import jax.numpy as jnp
from jax import lax
from jax.experimental import pallas as pl
from jax.experimental.pallas import tpu as pltpu
from jax.experimental.pallas import tpu_sc as plsc

D = 64
BATCH = 4096
HIST = 50
NC, NS = 2, 16
NW = NC * NS
PB = BATCH // NW
K = 8
G = 4
NIT = PB // K

_mesh = plsc.VectorSubcoreMesh(core_axis_name="c", subcore_axis_name="s")


@functools.partial(
    pl.kernel,
    mesh=_mesh,
    out_type=jax.ShapeDtypeStruct((BATCH, HIST, D), jnp.float32),
    compiler_params=pltpu.CompilerParams(use_tc_tiling_on_sc=False),
    scratch_types=[
        pltpu.VMEM((PB, HIST), jnp.int32),
        pltpu.VMEM((K, HIST, D), jnp.float32),
        pltpu.SemaphoreType.DMA((K,)),
        pltpu.SemaphoreType.DMA((K,)),
    ],
)
def _emb_lookup(idx_hbm, table_hbm, out_hbm, idx_v, rows_v, gsem, wsem):
    wid = lax.axis_index("s") * NC + lax.axis_index("c")
    base = wid * PB
    pltpu.sync_copy(idx_hbm.at[pl.ds(base, PB)], idx_v)

    def gather(r, b):
        pltpu.async_copy(table_hbm.at[idx_v.at[r]], rows_v.at[b], gsem.at[b])

    def wait_gather(b):
        pltpu.make_async_copy(
            out_hbm.at[0], rows_v.at[b], gsem.at[b]
        ).wait()

    def drain_write(b):
        pltpu.make_async_copy(
            out_hbm.at[0], rows_v.at[b], wsem.at[b]
        ).wait()

    for u in range(G):
        gather(u, u)

    def body(it, carry):
        r0 = it * K
        for u in range(K):
            r = r0 + u
            wait_gather(u)
            pltpu.async_copy(rows_v.at[u], out_hbm.at[base + r], wsem.at[u])
            bf = (u + G) % K

            def refill():
                gather(r + G, bf)

            def drain_and_refill():
                drain_write(bf)
                gather(r + G, bf)

            if u < G:
                pl.when(it > 0)(drain_and_refill)
                pl.when(it == 0)(refill)
            else:
                pl.when(it < NIT - 1)(drain_and_refill)
        return carry

    lax.fori_loop(0, NIT, body, 0)
    for u in range(K):
        drain_write(u)


def kernel(inputs, table):
    return _emb_lookup(inputs.astype(jnp.int32), table)
